# baseline (speedup 1.0000x reference)
import jax
import jax.numpy as jnp
from jax.experimental import pallas as pl
from jax.experimental.pallas import tpu as pltpu

_N = 5000
_NP = 5120
_B = 512
_NB = _NP // _B
_T = 0.5


def _nms_kernel(drows_ref, scol_ref, out_ref, rankc_ref, srows_ref, sup_ref):
    f32 = jnp.float32
    sub_b = jax.lax.broadcasted_iota(jnp.int32, (_B, _B), 0)
    lane_b = jax.lax.broadcasted_iota(jnp.int32, (_B, _B), 1)
    diag_b = jnp.where(sub_b == lane_b, 1.0, 0.0).astype(f32)

    def row2col(v):
        return jnp.sum(diag_b * v, axis=1, keepdims=True)

    def col2row(v):
        return jnp.sum(diag_b * v, axis=0, keepdims=True)

    s_row = drows_ref[4:5, :]
    sub_np = jax.lax.broadcasted_iota(jnp.int32, (_B, _NP), 0)
    lane_np = jax.lax.broadcasted_iota(jnp.int32, (_B, _NP), 1)
    d_np = lane_np - sub_np

    def rank_body(bi, carry):
        base = bi * _B
        sb = scol_ref[pl.ds(base, _B), :]
        gt = s_row > sb
        eq = (s_row == sb) & (d_np < base)
        cnt = jnp.sum(jnp.where(gt | eq, 1.0, 0.0).astype(f32),
                      axis=1, keepdims=True)
        rankc_ref[pl.ds(base, _B), :] = cnt
        return carry

    jax.lax.fori_loop(0, _NB, rank_body, 0)

    bf16 = jnp.bfloat16
    lane_bc = jax.lax.broadcasted_iota(jnp.int32, (_NP, _B), 1)
    d0 = drows_ref[:, :]
    dh = d0.astype(bf16)
    r1 = d0 - dh.astype(f32)
    dm = r1.astype(bf16)
    dl = (r1 - dm.astype(f32)).astype(bf16)

    def perm_body(bi, carry):
        base = bi * _B
        onehot_t = jnp.where(rankc_ref[:, :] == (base + lane_bc).astype(f32),
                             1.0, 0.0).astype(bf16)
        g = (jnp.dot(dh, onehot_t, preferred_element_type=f32)
             + jnp.dot(dm, onehot_t, preferred_element_type=f32))
        g = g + jnp.dot(dl, onehot_t, preferred_element_type=f32)
        srows_ref[:, pl.ds(base, _B)] = g
        return carry

    jax.lax.fori_loop(0, _NB, perm_body, 0)

    sup_ref[:, :] = jnp.zeros((1, _NP), f32)

    for bi in range(_NB):
        base = bi * _B
        rx1 = srows_ref[0:1, base:base + _B]
        ry1 = srows_ref[1:2, base:base + _B]
        rx2 = srows_ref[2:3, base:base + _B]
        ry2 = srows_ref[3:4, base:base + _B]
        rarea = (rx2 - rx1) * (ry2 - ry1)

        bx1 = row2col(rx1)
        by1 = row2col(ry1)
        bx2 = row2col(rx2)
        by2 = row2col(ry2)
        barea = (bx2 - bx1) * (by2 - by1)

        wx = jnp.maximum(jnp.minimum(bx2, rx2) - jnp.maximum(bx1, rx1), 0.0)
        wy = jnp.maximum(jnp.minimum(by2, ry2) - jnp.maximum(by1, ry1), 0.0)
        inter = wx * wy
        iou = inter / jnp.maximum(barea + rarea - inter, 1e-9)
        m = jnp.where((iou > _T) & (sub_b < lane_b), 1.0, 0.0).astype(f32)

        ext_row = 1.0 - sup_ref[0:1, base:base + _B]
        ext_col = row2col(ext_row)

        def fp_cond(c):
            return c[2]

        def fp_body(c):
            kc, kr, _ = c
            s = jnp.max(m * kc, axis=0, keepdims=True)
            kr2 = ext_row * (1.0 - s)
            kc2 = row2col(kr2)
            changed = jnp.max(jnp.abs(kr2 - kr)) > 0.0
            return (kc2, kr2, changed)

        keep_col, keep_row, _ = jax.lax.while_loop(
            fp_cond, fp_body, (ext_col, ext_row, jnp.array(True)))

        out_ref[:, base:base + _B] = srows_ref[:, base:base + _B] * keep_row

        rest = _NP - base - _B
        if rest == 0:
            continue
        lo = base + _B
        gx1 = srows_ref[0:1, lo:]
        gy1 = srows_ref[1:2, lo:]
        gx2 = srows_ref[2:3, lo:]
        gy2 = srows_ref[3:4, lo:]
        garea = (gx2 - gx1) * (gy2 - gy1)
        cwx = jnp.maximum(jnp.minimum(bx2, gx2) - jnp.maximum(bx1, gx1), 0.0)
        cwy = jnp.maximum(jnp.minimum(by2, gy2) - jnp.maximum(by1, gy1), 0.0)
        cinter = cwx * cwy
        ciou = cinter / jnp.maximum(barea + garea - cinter, 1e-9)
        hit = jnp.where(ciou > _T, keep_col, 0.0)
        sup_new = jnp.max(hit, axis=0, keepdims=True)
        sup_ref[0:1, lo:] = jnp.maximum(sup_ref[0:1, lo:], sup_new)


def kernel(boxes, scores):
    b = jnp.zeros((_NP, 4), jnp.float32).at[:_N].set(boxes.astype(jnp.float32))
    s = jnp.full((_NP,), -1.0, jnp.float32).at[:_N].set(
        scores.astype(jnp.float32))
    drows = jnp.concatenate(
        [b, s[:, None], jnp.zeros((_NP, 3), jnp.float32)], axis=1).T
    scol = s[:, None]
    out = pl.pallas_call(
        _nms_kernel,
        out_shape=jax.ShapeDtypeStruct((8, _NP), jnp.float32),
        scratch_shapes=[
            pltpu.VMEM((_NP, 1), jnp.float32),
            pltpu.VMEM((8, _NP), jnp.float32),
            pltpu.VMEM((1, _NP), jnp.float32),
        ],
    )(drows, scol)
    return out.T[:_N, :5]


# speedup vs baseline: 1.1574x; 1.1574x over previous
import jax
import jax.numpy as jnp
from jax.experimental import pallas as pl
from jax.experimental.pallas import tpu as pltpu

_N = 5000
_NP = 5120
_B = 512
_NB = _NP // _B
_T = 0.5


def _nms_kernel(drows_ref, scol_ref, out_ref, rankc_ref, srows_ref, sup_ref):
    f32 = jnp.float32
    sub_b = jax.lax.broadcasted_iota(jnp.int32, (_B, _B), 0)
    lane_b = jax.lax.broadcasted_iota(jnp.int32, (_B, _B), 1)
    diag_b = jnp.where(sub_b == lane_b, 1.0, 0.0).astype(f32)

    def row2col(v):
        return jnp.sum(diag_b * v, axis=1, keepdims=True)

    def col2row(v):
        return jnp.sum(diag_b * v, axis=0, keepdims=True)

    s_row = drows_ref[4:5, :]
    sub_np = jax.lax.broadcasted_iota(jnp.int32, (_B, _NP), 0)
    lane_np = jax.lax.broadcasted_iota(jnp.int32, (_B, _NP), 1)
    d_np = lane_np - sub_np

    def rank_body(bi, carry):
        base = bi * _B
        sb = scol_ref[pl.ds(base, _B), :]
        before = (s_row > sb) | ((s_row == sb) & (d_np < base))
        cnt = jnp.sum(jnp.where(before, 1.0, 0.0).astype(f32),
                      axis=1, keepdims=True)
        rankc_ref[pl.ds(base, _B), :] = cnt
        return carry

    jax.lax.fori_loop(0, _NB, rank_body, 0)

    bf16 = jnp.bfloat16
    lane_bc = jax.lax.broadcasted_iota(jnp.int32, (_NP, _B), 1)
    d0 = drows_ref[:, :]
    dh = d0.astype(bf16)
    r1 = d0 - dh.astype(f32)
    dm = r1.astype(bf16)
    dl = (r1 - dm.astype(f32)).astype(bf16)

    dstack = jnp.concatenate([dh, dm, dl], axis=0)

    def perm_body(bi, carry):
        base = bi * _B
        onehot_t = jnp.where(rankc_ref[:, :] == (base + lane_bc).astype(f32),
                             1.0, 0.0).astype(bf16)
        g24 = jnp.dot(dstack, onehot_t, preferred_element_type=f32)
        g = (g24[0:8, :] + g24[8:16, :]) + g24[16:24, :]
        srows_ref[:, pl.ds(base, _B)] = g
        return carry

    jax.lax.fori_loop(0, _NB, perm_body, 0)

    sup_ref[:, :] = jnp.zeros((1, _NP), f32)

    for bi in range(_NB):
        base = bi * _B
        rx1 = srows_ref[0:1, base:base + _B]
        ry1 = srows_ref[1:2, base:base + _B]
        rx2 = srows_ref[2:3, base:base + _B]
        ry2 = srows_ref[3:4, base:base + _B]
        rarea = (rx2 - rx1) * (ry2 - ry1)

        bx1 = row2col(rx1)
        by1 = row2col(ry1)
        bx2 = row2col(rx2)
        by2 = row2col(ry2)
        barea = (bx2 - bx1) * (by2 - by1)

        wx = jnp.maximum(jnp.minimum(bx2, rx2) - jnp.maximum(bx1, rx1), 0.0)
        wy = jnp.maximum(jnp.minimum(by2, ry2) - jnp.maximum(by1, ry1), 0.0)
        inter = wx * wy
        iou = inter / (barea + rarea - inter)
        m = jnp.where((iou > _T) & (sub_b < lane_b), 1.0, 0.0).astype(f32)

        ext_row = 1.0 - sup_ref[0:1, base:base + _B]
        ext_col = row2col(ext_row)

        def fp_cond(c):
            return c[2]

        def fp_body(c):
            kc, kr, _ = c
            s = jnp.max(m * kc, axis=0, keepdims=True)
            kr2 = ext_row * (1.0 - s)
            kc2 = row2col(kr2)
            changed = jnp.max(jnp.abs(kr2 - kr)) > 0.0
            return (kc2, kr2, changed)

        keep_col, keep_row, _ = jax.lax.while_loop(
            fp_cond, fp_body, (ext_col, ext_row, jnp.array(True)))

        out_ref[:, base:base + _B] = srows_ref[:, base:base + _B] * keep_row

        rest = _NP - base - _B
        if rest == 0:
            continue
        lo = base + _B
        gx1 = srows_ref[0:1, lo:]
        gy1 = srows_ref[1:2, lo:]
        gx2 = srows_ref[2:3, lo:]
        gy2 = srows_ref[3:4, lo:]
        garea = (gx2 - gx1) * (gy2 - gy1)
        cwx = jnp.maximum(jnp.minimum(bx2, gx2) - jnp.maximum(bx1, gx1), 0.0)
        cwy = jnp.maximum(jnp.minimum(by2, gy2) - jnp.maximum(by1, gy1), 0.0)
        cinter = cwx * cwy
        ciou = cinter / (barea + garea - cinter)
        hit = jnp.where(ciou > _T, keep_col, 0.0)
        sup_new = jnp.max(hit, axis=0, keepdims=True)
        sup_ref[0:1, lo:] = jnp.maximum(sup_ref[0:1, lo:], sup_new)


def kernel(boxes, scores):
    b = jnp.zeros((_NP, 4), jnp.float32).at[:_N].set(boxes.astype(jnp.float32))
    s = jnp.full((_NP,), -1.0, jnp.float32).at[:_N].set(
        scores.astype(jnp.float32))
    drows = jnp.concatenate(
        [b, s[:, None], jnp.zeros((_NP, 3), jnp.float32)], axis=1).T
    scol = s[:, None]
    out = pl.pallas_call(
        _nms_kernel,
        out_shape=jax.ShapeDtypeStruct((8, _NP), jnp.float32),
        scratch_shapes=[
            pltpu.VMEM((_NP, 1), jnp.float32),
            pltpu.VMEM((8, _NP), jnp.float32),
            pltpu.VMEM((1, _NP), jnp.float32),
        ],
    )(drows, scol)
    return out.T[:_N, :5]
